# baseline (device time: 68245 ns/iter reference)
import functools

import jax
import jax.numpy as jnp
from jax import lax
from jax.experimental import pallas as pl
from jax.experimental.pallas import tpu as pltpu

N_DEV = 32
V_PER = 4096
T = 512
D = 512
CAP = 32


def kernel(ids, E):
    my = lax.axis_index("i")
    owner = ids // V_PER
    local = ids - my * V_PER

    idx_all = jax.vmap(
        lambda s: jnp.nonzero(owner == s, size=CAP, fill_value=T)[0]
    )(jnp.arange(N_DEV)).astype(jnp.int32)
    idx_mine = idx_all[my]
    rows = E[
        jnp.clip(local[jnp.clip(idx_mine, 0, T - 1)], 0, V_PER - 1)
    ]
    idx_flat = idx_all.reshape((1, N_DEV * CAP))

    def body(rows_ref, idxf_ref, o_ref, stage_ref, send_sem, recv_sem):
        me = lax.axis_index("i")

        barrier = pltpu.get_barrier_semaphore()
        for d in range(1, N_DEV):
            pl.semaphore_signal(
                barrier, inc=1,
                device_id=((me + d) % N_DEV,),
                device_id_type=pl.DeviceIdType.MESH,
            )
        pl.semaphore_wait(barrier, N_DEV - 1)

        my_slot = pl.multiple_of(me * CAP, CAP)
        stage_ref[pl.ds(my_slot, CAP)] = rows_ref[...]
        for d in range(1, N_DEV):
            pltpu.make_async_remote_copy(
                src_ref=rows_ref,
                dst_ref=stage_ref.at[pl.ds(my_slot, CAP)],
                send_sem=send_sem,
                recv_sem=recv_sem,
                device_id=((me + d) % N_DEV,),
                device_id_type=pl.DeviceIdType.MESH,
            ).start()

        waiter = pltpu.make_async_remote_copy(
            src_ref=rows_ref,
            dst_ref=stage_ref.at[pl.ds(0, CAP)],
            send_sem=send_sem,
            recv_sem=recv_sem,
            device_id=(me,),
            device_id_type=pl.DeviceIdType.MESH,
        )
        for _ in range(N_DEV - 1):
            waiter.wait_recv()
        for _ in range(N_DEV - 1):
            waiter.wait_send()

        tok = lax.broadcasted_iota(jnp.int32, (T, N_DEV * CAP), 0)
        p = (tok == idxf_ref[...]).astype(jnp.float32)
        o_ref[...] = jnp.dot(
            p, stage_ref[...], preferred_element_type=jnp.float32
        )

        @functools.partial(pl.run_scoped, sem2=pltpu.SemaphoreType.REGULAR)
        def _(sem2):
            for d in range(1, N_DEV):
                pl.semaphore_signal(
                    sem2, inc=1,
                    device_id=((me + d) % N_DEV,),
                    device_id_type=pl.DeviceIdType.MESH,
                )
            pl.semaphore_wait(sem2, N_DEV - 1)

    return pl.pallas_call(
        body,
        out_shape=jax.ShapeDtypeStruct((T, D), jnp.float32),
        in_specs=[
            pl.BlockSpec(memory_space=pltpu.VMEM),
            pl.BlockSpec(memory_space=pltpu.VMEM),
        ],
        out_specs=pl.BlockSpec(memory_space=pltpu.VMEM),
        scratch_shapes=[
            pltpu.VMEM((N_DEV * CAP, D), jnp.float32),
            pltpu.SemaphoreType.DMA,
            pltpu.SemaphoreType.DMA,
        ],
        compiler_params=pltpu.CompilerParams(collective_id=0),
    )(rows, idx_flat)


# device time: 45099 ns/iter; 1.5132x vs baseline; 1.5132x over previous
import functools

import jax
import jax.numpy as jnp
from jax import lax
from jax.experimental import pallas as pl
from jax.experimental.pallas import tpu as pltpu

N_DEV = 32
V_PER = 4096
T = 512
D = 512
CAP = 32


def kernel(ids, E):
    my = lax.axis_index("i")
    owner = ids // V_PER
    local = ids - my * V_PER
    owned = owner == my
    partial = jnp.where(owned[:, None], E[jnp.clip(local, 0, V_PER - 1)], 0.0)

    onehot = (owner[:, None] == jnp.arange(N_DEV)[None, :]).astype(jnp.int32)
    rank = (jnp.cumsum(onehot, axis=0) * onehot).sum(axis=1) - 1
    col = (owner * CAP + rank).astype(jnp.int32).reshape((T, 1))

    def body(x_ref, col_ref, o_ref, stage_ref, send_sem, recv_sem):
        me = lax.axis_index("i")

        barrier = pltpu.get_barrier_semaphore()
        for d in range(1, N_DEV):
            pl.semaphore_signal(
                barrier, inc=1,
                device_id=((me + d) % N_DEV,),
                device_id_type=pl.DeviceIdType.MESH,
            )
        pl.semaphore_wait(barrier, N_DEV - 1)

        sel = (
            col_ref[...] == lax.broadcasted_iota(jnp.int32, (T, CAP), 1)
            + me * CAP
        ).astype(jnp.float32)
        rows = jnp.dot(sel.T, x_ref[...], preferred_element_type=jnp.float32)

        my_slot = pl.multiple_of(me * CAP, CAP)
        stage_ref[pl.ds(my_slot, CAP)] = rows
        for d in range(1, N_DEV):
            pltpu.make_async_remote_copy(
                src_ref=stage_ref.at[pl.ds(my_slot, CAP)],
                dst_ref=stage_ref.at[pl.ds(my_slot, CAP)],
                send_sem=send_sem,
                recv_sem=recv_sem,
                device_id=((me + d) % N_DEV,),
                device_id_type=pl.DeviceIdType.MESH,
            ).start()

        waiter = pltpu.make_async_remote_copy(
            src_ref=stage_ref.at[pl.ds(0, CAP)],
            dst_ref=stage_ref.at[pl.ds(0, CAP)],
            send_sem=send_sem,
            recv_sem=recv_sem,
            device_id=(me,),
            device_id_type=pl.DeviceIdType.MESH,
        )
        for _ in range(N_DEV - 1):
            waiter.wait_recv()
        for _ in range(N_DEV - 1):
            waiter.wait_send()

        p = (
            col_ref[...]
            == lax.broadcasted_iota(jnp.int32, (T, N_DEV * CAP), 1)
        ).astype(jnp.float32)
        o_ref[...] = jnp.dot(
            p, stage_ref[...], preferred_element_type=jnp.float32
        )

        @functools.partial(pl.run_scoped, sem2=pltpu.SemaphoreType.REGULAR)
        def _(sem2):
            for d in range(1, N_DEV):
                pl.semaphore_signal(
                    sem2, inc=1,
                    device_id=((me + d) % N_DEV,),
                    device_id_type=pl.DeviceIdType.MESH,
                )
            pl.semaphore_wait(sem2, N_DEV - 1)

    return pl.pallas_call(
        body,
        out_shape=jax.ShapeDtypeStruct((T, D), jnp.float32),
        in_specs=[
            pl.BlockSpec(memory_space=pltpu.VMEM),
            pl.BlockSpec(memory_space=pltpu.VMEM),
        ],
        out_specs=pl.BlockSpec(memory_space=pltpu.VMEM),
        scratch_shapes=[
            pltpu.VMEM((N_DEV * CAP, D), jnp.float32),
            pltpu.SemaphoreType.DMA,
            pltpu.SemaphoreType.DMA,
        ],
        compiler_params=pltpu.CompilerParams(collective_id=0),
    )(partial, col)


# device time: 33314 ns/iter; 2.0485x vs baseline; 1.3538x over previous
import functools

import jax
import jax.numpy as jnp
from jax import lax
from jax.experimental import pallas as pl
from jax.experimental.pallas import tpu as pltpu

N_DEV = 32
V_PER = 4096
T = 512
D = 512
CAP = 32


def kernel(ids, E):
    my = lax.axis_index("i")
    owner = ids // V_PER
    local = ids - my * V_PER
    owned = owner == my
    partial = jnp.where(
        owned[:, None], E[jnp.clip(local, 0, V_PER - 1)], 0.0
    ).astype(jnp.bfloat16)

    onehot = (owner[:, None] == jnp.arange(N_DEV)[None, :]).astype(jnp.int32)
    rank = (jnp.cumsum(onehot, axis=0) * onehot).sum(axis=1) - 1
    col = (owner * CAP + rank).astype(jnp.int32).reshape((T, 1))

    def body(x_ref, col_ref, o_ref, stage_ref, send_sem, recv_sem):
        me = lax.axis_index("i")

        barrier = pltpu.get_barrier_semaphore()
        for d in range(1, N_DEV):
            pl.semaphore_signal(
                barrier, inc=1,
                device_id=((me + d) % N_DEV,),
                device_id_type=pl.DeviceIdType.MESH,
            )
        pl.semaphore_wait(barrier, N_DEV - 1)

        sel = (
            col_ref[...] == lax.broadcasted_iota(jnp.int32, (T, CAP), 1)
            + me * CAP
        ).astype(jnp.bfloat16)
        rows = jnp.dot(sel.T, x_ref[...], preferred_element_type=jnp.float32)

        my_slot = pl.multiple_of(me * CAP, CAP)
        stage_ref[pl.ds(my_slot, CAP)] = rows.astype(jnp.bfloat16)
        for d in range(1, N_DEV):
            pltpu.make_async_remote_copy(
                src_ref=stage_ref.at[pl.ds(my_slot, CAP)],
                dst_ref=stage_ref.at[pl.ds(my_slot, CAP)],
                send_sem=send_sem,
                recv_sem=recv_sem,
                device_id=((me + d) % N_DEV,),
                device_id_type=pl.DeviceIdType.MESH,
            ).start()

        waiter = pltpu.make_async_remote_copy(
            src_ref=stage_ref.at[pl.ds(0, CAP)],
            dst_ref=stage_ref.at[pl.ds(0, CAP)],
            send_sem=send_sem,
            recv_sem=recv_sem,
            device_id=(me,),
            device_id_type=pl.DeviceIdType.MESH,
        )
        for _ in range(N_DEV - 1):
            waiter.wait_recv()
        for _ in range(N_DEV - 1):
            waiter.wait_send()

        p = (
            col_ref[...]
            == lax.broadcasted_iota(jnp.int32, (T, N_DEV * CAP), 1)
        ).astype(jnp.bfloat16)
        o_ref[...] = jnp.dot(
            p, stage_ref[...], preferred_element_type=jnp.float32
        )

        @functools.partial(pl.run_scoped, sem2=pltpu.SemaphoreType.REGULAR)
        def _(sem2):
            for d in range(1, N_DEV):
                pl.semaphore_signal(
                    sem2, inc=1,
                    device_id=((me + d) % N_DEV,),
                    device_id_type=pl.DeviceIdType.MESH,
                )
            pl.semaphore_wait(sem2, N_DEV - 1)

    return pl.pallas_call(
        body,
        out_shape=jax.ShapeDtypeStruct((T, D), jnp.float32),
        in_specs=[
            pl.BlockSpec(memory_space=pltpu.VMEM),
            pl.BlockSpec(memory_space=pltpu.VMEM),
        ],
        out_specs=pl.BlockSpec(memory_space=pltpu.VMEM),
        scratch_shapes=[
            pltpu.VMEM((N_DEV * CAP, D), jnp.bfloat16),
            pltpu.SemaphoreType.DMA,
            pltpu.SemaphoreType.DMA,
        ],
        compiler_params=pltpu.CompilerParams(collective_id=0),
    )(partial, col)


# device time: 29639 ns/iter; 2.3025x vs baseline; 1.1240x over previous
import functools

import jax
import jax.numpy as jnp
from jax import lax
from jax.experimental import pallas as pl
from jax.experimental.pallas import tpu as pltpu

N_DEV = 32
V_PER = 4096
T = 512
D = 512
CAP = 32
CHUNK = 8
N_CHUNKS = CAP // CHUNK


def kernel(ids, E):
    my = lax.axis_index("i")
    owner = ids // V_PER
    local = ids - my * V_PER
    owned = owner == my
    partial = jnp.where(
        owned[:, None], E[jnp.clip(local, 0, V_PER - 1)], 0.0
    ).astype(jnp.bfloat16)

    onehot = (owner[:, None] == jnp.arange(N_DEV)[None, :]).astype(jnp.int32)
    counts = onehot.sum(axis=0).reshape((1, N_DEV))
    rank = (jnp.cumsum(onehot, axis=0) * onehot).sum(axis=1) - 1
    col = (owner * CAP + rank).astype(jnp.int32).reshape((T, 1))

    def body(x_ref, col_ref, cnt_ref, o_ref, stage_ref, send_sem, recv_sem):
        me = lax.axis_index("i")

        stage_ref[...] = jnp.zeros((N_DEV * CAP, D), jnp.bfloat16)

        sel = (
            col_ref[...] == lax.broadcasted_iota(jnp.int32, (T, CAP), 1)
            + me * CAP
        ).astype(jnp.bfloat16)
        rows = jnp.dot(sel.T, x_ref[...], preferred_element_type=jnp.float32)
        my_slot = pl.multiple_of(me * CAP, CAP)
        stage_ref[pl.ds(my_slot, CAP)] = rows.astype(jnp.bfloat16)

        barrier = pltpu.get_barrier_semaphore()
        for d in range(1, N_DEV):
            pl.semaphore_signal(
                barrier, inc=1,
                device_id=((me + d) % N_DEV,),
                device_id_type=pl.DeviceIdType.MESH,
            )
        pl.semaphore_wait(barrier, N_DEV - 1)

        c_me = cnt_ref[0, me]
        for k in range(N_CHUNKS):
            @pl.when(k * CHUNK < c_me)
            def _():
                src = stage_ref.at[
                    pl.ds(pl.multiple_of(my_slot + k * CHUNK, CHUNK), CHUNK)
                ]
                for d in range(1, N_DEV):
                    pltpu.make_async_remote_copy(
                        src_ref=src,
                        dst_ref=src,
                        send_sem=send_sem,
                        recv_sem=recv_sem,
                        device_id=((me + d) % N_DEV,),
                        device_id_type=pl.DeviceIdType.MESH,
                    ).start()

        waiter = pltpu.make_async_remote_copy(
            src_ref=stage_ref.at[pl.ds(0, CHUNK)],
            dst_ref=stage_ref.at[pl.ds(0, CHUNK)],
            send_sem=send_sem,
            recv_sem=recv_sem,
            device_id=(me,),
            device_id_type=pl.DeviceIdType.MESH,
        )
        for d in range(1, N_DEV):
            c_s = cnt_ref[0, (me + d) % N_DEV]
            for k in range(N_CHUNKS):
                @pl.when(k * CHUNK < c_s)
                def _():
                    waiter.wait_recv()
        for k in range(N_CHUNKS):
            @pl.when(k * CHUNK < c_me)
            def _():
                for _ in range(N_DEV - 1):
                    waiter.wait_send()

        p = (
            col_ref[...]
            == lax.broadcasted_iota(jnp.int32, (T, N_DEV * CAP), 1)
        ).astype(jnp.bfloat16)
        o_ref[...] = jnp.dot(
            p, stage_ref[...], preferred_element_type=jnp.float32
        )

        @functools.partial(pl.run_scoped, sem2=pltpu.SemaphoreType.REGULAR)
        def _(sem2):
            for d in range(1, N_DEV):
                pl.semaphore_signal(
                    sem2, inc=1,
                    device_id=((me + d) % N_DEV,),
                    device_id_type=pl.DeviceIdType.MESH,
                )
            pl.semaphore_wait(sem2, N_DEV - 1)

    return pl.pallas_call(
        body,
        out_shape=jax.ShapeDtypeStruct((T, D), jnp.float32),
        in_specs=[
            pl.BlockSpec(memory_space=pltpu.VMEM),
            pl.BlockSpec(memory_space=pltpu.VMEM),
            pl.BlockSpec(memory_space=pltpu.SMEM),
        ],
        out_specs=pl.BlockSpec(memory_space=pltpu.VMEM),
        scratch_shapes=[
            pltpu.VMEM((N_DEV * CAP, D), jnp.bfloat16),
            pltpu.SemaphoreType.DMA,
            pltpu.SemaphoreType.DMA,
        ],
        compiler_params=pltpu.CompilerParams(collective_id=0),
    )(partial, col, counts)


# device time: 23816 ns/iter; 2.8655x vs baseline; 1.2445x over previous
import functools

import jax
import jax.numpy as jnp
from jax import lax
from jax.experimental import pallas as pl
from jax.experimental.pallas import tpu as pltpu

N_DEV = 32
V_PER = 4096
T = 512
D = 512
CAP = 32
CHUNK = 8
N_CHUNKS = CAP // CHUNK


def kernel(ids, E):
    my = lax.axis_index("i")
    owner = ids // V_PER
    local = ids - my * V_PER
    partial = E[jnp.clip(local, 0, V_PER - 1)].astype(jnp.bfloat16)

    onehot = (owner[:, None] == jnp.arange(N_DEV)[None, :]).astype(jnp.int32)
    counts = onehot.sum(axis=0).reshape((1, N_DEV))
    rank = (jnp.cumsum(onehot, axis=0) * onehot).sum(axis=1) - 1
    col = (owner * CAP + rank).astype(jnp.int32).reshape((T, 1))

    def body(x_ref, col_ref, cnt_ref, o_ref, stage_ref, send_sem, recv_sem):
        me = lax.axis_index("i")

        stage_ref[...] = jnp.zeros((N_DEV * CAP, D), jnp.bfloat16)

        sel = (
            col_ref[...] == lax.broadcasted_iota(jnp.int32, (T, CAP), 1)
            + me * CAP
        ).astype(jnp.bfloat16)
        rows = jnp.dot(sel.T, x_ref[...], preferred_element_type=jnp.float32)
        my_slot = pl.multiple_of(me * CAP, CAP)
        stage_ref[pl.ds(my_slot, CAP)] = rows.astype(jnp.bfloat16)

        barrier = pltpu.get_barrier_semaphore()
        for d in range(1, N_DEV):
            pl.semaphore_signal(
                barrier, inc=1,
                device_id=((me + d) % N_DEV,),
                device_id_type=pl.DeviceIdType.MESH,
            )
        pl.semaphore_wait(barrier, N_DEV - 1)

        c_me = cnt_ref[0, me]
        for k in range(N_CHUNKS):
            def _send_chunk(k=k):
                src = stage_ref.at[
                    pl.ds(pl.multiple_of(my_slot + k * CHUNK, CHUNK), CHUNK)
                ]
                for d in range(1, N_DEV):
                    pltpu.make_async_remote_copy(
                        src_ref=src,
                        dst_ref=src,
                        send_sem=send_sem,
                        recv_sem=recv_sem,
                        device_id=((me + d) % N_DEV,),
                        device_id_type=pl.DeviceIdType.MESH,
                    ).start()

            if k == 0:
                _send_chunk()
            else:
                pl.when(k * CHUNK < c_me)(_send_chunk)

        waiter = pltpu.make_async_remote_copy(
            src_ref=stage_ref.at[pl.ds(0, CHUNK)],
            dst_ref=stage_ref.at[pl.ds(0, CHUNK)],
            send_sem=send_sem,
            recv_sem=recv_sem,
            device_id=(me,),
            device_id_type=pl.DeviceIdType.MESH,
        )
        for d in range(1, N_DEV):
            c_s = cnt_ref[0, (me + d) % N_DEV]
            waiter.wait_recv()
            for k in range(1, N_CHUNKS):
                @pl.when(k * CHUNK < c_s)
                def _():
                    waiter.wait_recv()
        for _ in range(N_DEV - 1):
            waiter.wait_send()
        for k in range(1, N_CHUNKS):
            @pl.when(k * CHUNK < c_me)
            def _():
                for _ in range(N_DEV - 1):
                    waiter.wait_send()

        p = (
            col_ref[...]
            == lax.broadcasted_iota(jnp.int32, (T, N_DEV * CAP), 1)
        ).astype(jnp.bfloat16)
        o_ref[...] = jnp.dot(
            p, stage_ref[...], preferred_element_type=jnp.float32
        )

    return pl.pallas_call(
        body,
        out_shape=jax.ShapeDtypeStruct((T, D), jnp.float32),
        in_specs=[
            pl.BlockSpec(memory_space=pltpu.VMEM),
            pl.BlockSpec(memory_space=pltpu.VMEM),
            pl.BlockSpec(memory_space=pltpu.SMEM),
        ],
        out_specs=pl.BlockSpec(memory_space=pltpu.VMEM),
        scratch_shapes=[
            pltpu.VMEM((N_DEV * CAP, D), jnp.bfloat16),
            pltpu.SemaphoreType.DMA,
            pltpu.SemaphoreType.DMA,
        ],
        compiler_params=pltpu.CompilerParams(collective_id=0),
    )(partial, col, counts)


# device time: 23733 ns/iter; 2.8755x vs baseline; 1.0035x over previous
import jax
import jax.numpy as jnp
from jax import lax
from jax.experimental import pallas as pl
from jax.experimental.pallas import tpu as pltpu

N_DEV = 32
V_PER = 4096
T = 512
D = 512
CAP = 32
CHUNK = 8
N_CHUNKS = CAP // CHUNK


def kernel(ids, E):
    my = lax.axis_index("i")
    owner = ids // V_PER
    local = ids - my * V_PER
    partial = E[jnp.clip(local, 0, V_PER - 1)].astype(jnp.bfloat16)

    onehot = (owner[:, None] == jnp.arange(N_DEV)[None, :]).astype(jnp.int32)
    counts = onehot.sum(axis=0).reshape((1, N_DEV))
    rank = (jnp.cumsum(onehot, axis=0) * onehot).sum(axis=1) - 1
    col = (owner * CAP + rank).astype(jnp.int32).reshape((T, 1))

    def body(x_ref, col_ref, cnt_ref, o_ref, stage_ref, send_sem, recv_sem):
        me = lax.axis_index("i")

        stage_ref[...] = jnp.zeros((N_DEV * CAP, D), jnp.bfloat16)

        sel = (
            col_ref[...] == lax.broadcasted_iota(jnp.int32, (T, CAP), 1)
            + me * CAP
        ).astype(jnp.bfloat16)
        rows = jnp.dot(sel.T, x_ref[...], preferred_element_type=jnp.float32)
        my_slot = pl.multiple_of(me * CAP, CAP)
        stage_ref[pl.ds(my_slot, CAP)] = rows.astype(jnp.bfloat16)

        barrier = pltpu.get_barrier_semaphore()
        for d in range(1, N_DEV):
            pl.semaphore_signal(
                barrier, inc=1,
                device_id=((me + d) % N_DEV,),
                device_id_type=pl.DeviceIdType.MESH,
            )
        pl.semaphore_wait(barrier, N_DEV - 1)

        c_me = cnt_ref[0, me]
        for k in range(N_CHUNKS):
            def _send_chunk(k=k):
                src = stage_ref.at[
                    pl.ds(pl.multiple_of(my_slot + k * CHUNK, CHUNK), CHUNK)
                ]
                for d in range(1, N_DEV):
                    pltpu.make_async_remote_copy(
                        src_ref=src,
                        dst_ref=src,
                        send_sem=send_sem,
                        recv_sem=recv_sem,
                        device_id=((me + d) % N_DEV,),
                        device_id_type=pl.DeviceIdType.MESH,
                    ).start()

            if k == 0:
                _send_chunk()
            else:
                pl.when(k * CHUNK < c_me)(_send_chunk)

        p = (
            col_ref[...]
            == lax.broadcasted_iota(jnp.int32, (T, N_DEV * CAP), 1)
        ).astype(jnp.bfloat16)

        waiter = pltpu.make_async_remote_copy(
            src_ref=stage_ref.at[pl.ds(0, CHUNK)],
            dst_ref=stage_ref.at[pl.ds(0, CHUNK)],
            send_sem=send_sem,
            recv_sem=recv_sem,
            device_id=(me,),
            device_id_type=pl.DeviceIdType.MESH,
        )
        for d in range(1, N_DEV):
            c_s = cnt_ref[0, (me + d) % N_DEV]
            waiter.wait_recv()
            for k in range(1, N_CHUNKS):
                @pl.when(k * CHUNK < c_s)
                def _():
                    waiter.wait_recv()
        for _ in range(N_DEV - 1):
            waiter.wait_send()
        for k in range(1, N_CHUNKS):
            @pl.when(k * CHUNK < c_me)
            def _():
                for _ in range(N_DEV - 1):
                    waiter.wait_send()

        o_ref[...] = jnp.dot(
            p, stage_ref[...], preferred_element_type=jnp.float32
        )

    return pl.pallas_call(
        body,
        out_shape=jax.ShapeDtypeStruct((T, D), jnp.float32),
        in_specs=[
            pl.BlockSpec(memory_space=pltpu.VMEM),
            pl.BlockSpec(memory_space=pltpu.VMEM),
            pl.BlockSpec(memory_space=pltpu.SMEM),
        ],
        out_specs=pl.BlockSpec(memory_space=pltpu.VMEM),
        scratch_shapes=[
            pltpu.VMEM((N_DEV * CAP, D), jnp.bfloat16),
            pltpu.SemaphoreType.DMA,
            pltpu.SemaphoreType.DMA,
        ],
        compiler_params=pltpu.CompilerParams(collective_id=0),
    )(partial, col, counts)
